# baseline (speedup 1.0000x reference)
import math

import jax
import jax.numpy as jnp
from jax.experimental import pallas as pl
from jax.experimental.pallas import tpu as pltpu

_D_MODEL = 96
_D_STATE = 16
_D_INNER = 96
_DT_RANK = 6
_KG = 4
_TOPK = 0.15
_SIZE = 16
_LN_EPS = 1e-5

_T = 64


def _silu(v):
    return v * jax.nn.sigmoid(v)


def _conv2d(v, w, b=None):
    out = jax.lax.conv_general_dilated(
        v, w, (1, 1), "SAME", dimension_numbers=("NCHW", "OIHW", "NCHW"))
    if b is not None:
        out = out + b[None, :, None, None]
    return out


def _scan_all_body(Bsz, T, N, d):
    G = 2 * Bsz * 2

    def body(xf_ref, xb_ref, wm_ref, wb_ref, wc_ref, a_ref, bias_ref, rev_ref,
             yf_ref, yb_ref, h_ref, da_ref, hb_ref):
        c = pl.program_id(0)

        @pl.when(c == 0)
        def _():
            h_ref[...] = jnp.zeros_like(h_ref)

        cc_all = [None] * G
        for dirn in range(2):
            x_ref = xf_ref if dirn == 0 else xb_ref
            for b in range(Bsz):
                X = x_ref[b]
                if dirn == 1:
                    X = jnp.dot(rev_ref[...], X,
                                preferred_element_type=jnp.float32)
                for j in range(2):
                    k = dirn * 2 + j
                    g = (dirn * Bsz + b) * 2 + j
                    sl = slice(g * N, (g + 1) * N)
                    raw = jnp.dot(X, wm_ref[k],
                                  preferred_element_type=jnp.float32)
                    zb = raw + bias_ref[k]
                    delta = (jnp.maximum(zb, 0.0)
                             + jnp.log1p(jnp.exp(-jnp.abs(zb))))
                    Bc = jnp.dot(X, wb_ref[k],
                                 preferred_element_type=jnp.float32)
                    Cc = jnp.dot(X, wc_ref[k],
                                 preferred_element_type=jnp.float32)
                    Gx = delta * X
                    da_ref[:, sl, :] = jnp.exp(
                        delta[:, None, :] * a_ref[k][None, :, :])
                    hb_ref[:, sl, :] = Bc[:, :, None] * Gx[:, None, :]
                    cc_all[g] = Cc

        def step(t, h):
            h = h * da_ref[t] + hb_ref[t]
            hb_ref[t] = h
            return h

        h_ref[...] = jax.lax.fori_loop(0, T, step, h_ref[...])

        for dirn in range(2):
            y_ref = yf_ref if dirn == 0 else yb_ref
            for b in range(Bsz):
                y = jnp.zeros((T, d), jnp.float32)
                for j in range(2):
                    g = (dirn * Bsz + b) * 2 + j
                    sl = slice(g * N, (g + 1) * N)
                    y = y + jnp.sum(hb_ref[:, sl, :]
                                    * cc_all[g][:, :, None], axis=1)
                if dirn == 1:
                    y = jnp.dot(rev_ref[...], y,
                                preferred_element_type=jnp.float32)
                y_ref[b] = y

    return body


def _scan_all(vals, M4, WB4, WC4, A4, bias4):
    Bsz, Lp, d = vals.shape
    T = _T
    NC = Lp // T
    N = _D_STATE
    G = 2 * Bsz * 2

    def fmap(c):
        return (0, c, 0)

    def bmap(c):
        return (0, NC - 1 - c, 0)

    def wmap(c):
        return (0, 0, 0)

    yf, yb = pl.pallas_call(
        _scan_all_body(Bsz, T, N, d),
        grid=(NC,),
        in_specs=[
            pl.BlockSpec((Bsz, T, d), fmap),
            pl.BlockSpec((Bsz, T, d), bmap),
            pl.BlockSpec((4, d, d), wmap),
            pl.BlockSpec((4, d, N), wmap),
            pl.BlockSpec((4, d, N), wmap),
            pl.BlockSpec((4, N, d), wmap),
            pl.BlockSpec((4, 1, d), wmap),
            pl.BlockSpec((T, T), lambda c: (0, 0)),
        ],
        out_specs=[
            pl.BlockSpec((Bsz, T, d), fmap),
            pl.BlockSpec((Bsz, T, d), bmap),
        ],
        out_shape=[
            jax.ShapeDtypeStruct((Bsz, Lp, d), jnp.float32),
            jax.ShapeDtypeStruct((Bsz, Lp, d), jnp.float32),
        ],
        scratch_shapes=[
            pltpu.VMEM((G * N, d), jnp.float32),
            pltpu.VMEM((T, G * N, d), jnp.float32),
            pltpu.VMEM((T, G * N, d), jnp.float32),
        ],
        compiler_params=pltpu.CompilerParams(
            dimension_semantics=("arbitrary",)),
    )(vals, vals, M4, WB4, WC4, A4, bias4,
      jnp.eye(T, dtype=jnp.float32)[::-1])
    return yf, yb


def _select_mask(xp):
    B, C, H, W = xp.shape
    gh, gw = H // _SIZE, W // _SIZE
    density = jax.nn.sigmoid(xp.mean(axis=1))
    density = density.reshape(B, gh, _SIZE, gw, _SIZE).mean(axis=(2, 4))
    sf = density.reshape(B, gh * gw)
    ths = []
    th = 0.3
    while th >= 0:
        ths.append(th)
        th = round(th - 0.05, 2)
    thr_arr = jnp.asarray(ths, dtype=jnp.float32)
    anys = sf[:, None, :] > thr_arr[None, :, None]
    anys = anys.any(axis=2)
    found = anys.any(axis=1)
    th_sel = thr_arr[jnp.argmax(anys, axis=1)]
    m = (sf > th_sel[:, None]).astype(jnp.float32)
    top10 = jax.lax.top_k(sf, 10)[0][:, -1]
    m_cap = (sf >= top10[:, None]).astype(jnp.float32)
    m = jnp.where(m.sum(axis=1, keepdims=True) > 10, m_cap, m)
    fallback = (sf >= sf.max(axis=1, keepdims=True)).astype(jnp.float32)
    return jnp.where(found[:, None], m, fallback)


def _build_indices(mask_cells, H, W, max_n):
    B = mask_cells.shape[0]
    gh, gw = H // _SIZE, W // _SIZE
    sel = mask_cells.reshape(B, gh, gw)
    sel = jnp.repeat(jnp.repeat(sel, _SIZE, axis=1), _SIZE, axis=2)
    sel = sel.reshape(B, H * W) > 0
    HW = H * W
    ar = jnp.arange(HW, dtype=jnp.int32)
    csum = jnp.cumsum(sel.astype(jnp.int32), axis=1)
    total = csum[:, -1]
    keep = sel & (csum <= max_n)
    keep = jnp.where((total > 0)[:, None], keep, (ar < 16)[None, :])
    kcs = jnp.cumsum(keep.astype(jnp.int32), axis=1)
    n_valid = kcs[:, -1]
    rank = jnp.where(keep, kcs - 1, n_valid[:, None] + ar[None, :] - kcs)

    def scat_one(rank_b):
        return jnp.zeros((max_n,), jnp.int32).at[rank_b].set(ar, mode="drop")

    idx = jax.vmap(scat_one)(rank)
    return idx, rank, keep, n_valid


def kernel(x, in_w, in_b, x_proj_weight, dt_projs_weight, dt_projs_bias,
           A_logs, Ds, proj_d_w, ln_g, ln_b, out_w):
    xact = _silu(_conv2d(x, in_w, in_b))
    xp, z = jnp.split(xact, 2, axis=1)
    B, D, H, W = xp.shape
    HW = H * W
    max_n = int(HW * _TOPK)
    Lp = ((max_n + _T - 1) // _T) * _T

    mask_cells = _select_mask(xp)
    idx, rank, keep, n_valid = _build_indices(mask_cells, H, W, max_n)
    validp = jnp.arange(Lp, dtype=jnp.int32)[None, :] < n_valid[:, None]

    xs_flat = xp.reshape(B, D, HW)
    idx_pad = jnp.pad(idx, ((0, 0), (0, Lp - max_n)))
    vals = jnp.take_along_axis(xs_flat, idx_pad[:, None, :], axis=2)
    vals = vals * validp[:, None, :].astype(vals.dtype)
    vals_t = jnp.swapaxes(vals, 1, 2)

    Ds_full = _silu(Ds @ proj_d_w.T)
    Ds_t = Ds_full[:_KG * _D_INNER].reshape(_KG, _D_INNER)
    Ds_b = Ds_full[_KG * _D_INNER:]
    dsum = Ds_t.sum(axis=0)

    W6 = x_proj_weight[:, :_DT_RANK, :]
    WB = x_proj_weight[:, _DT_RANK:_DT_RANK + _D_STATE, :]
    WC = x_proj_weight[:, _DT_RANK + _D_STATE:, :]
    M = jnp.einsum("krd,ker->kde", W6, dt_projs_weight)
    WBt = jnp.swapaxes(WB, 1, 2)
    WCt = jnp.swapaxes(WC, 1, 2)
    A = -jnp.exp(A_logs).reshape(_KG, _D_INNER, _D_STATE)
    At = jnp.swapaxes(A, 1, 2)
    bias = dt_projs_bias.reshape(_KG, 1, _D_INNER)

    y_f, y_b = _scan_all(vals_t, M, WBt, WCt, At, bias)

    y_sum = (y_f + y_b + vals_t * dsum[None, None, :]) * 0.5
    y_comb = jnp.swapaxes(y_sum, 1, 2)

    rank_c = jnp.minimum(rank, Lp - 1)
    ygr = jnp.take_along_axis(y_comb, rank_c[:, None, :], axis=2)
    base = Ds_b[None, :, None] * xs_flat
    out_flat = jnp.where(keep[:, None, :], ygr, base)
    out_y = out_flat.reshape(B, D, H, W)

    mu = out_y.mean(axis=1, keepdims=True)
    var = ((out_y - mu) ** 2).mean(axis=1, keepdims=True)
    oy = (out_y - mu) * jax.lax.rsqrt(var + _LN_EPS)
    oy = oy * ln_g[None, :, None, None] + ln_b[None, :, None, None]
    y = oy * z
    return _conv2d(y, out_w, None)


# speedup vs baseline: 3.1396x; 3.1396x over previous
import math

import jax
import jax.numpy as jnp
from jax.experimental import pallas as pl
from jax.experimental.pallas import tpu as pltpu

_D_MODEL = 96
_D_STATE = 16
_D_INNER = 96
_DT_RANK = 6
_KG = 4
_TOPK = 0.15
_SIZE = 16
_LN_EPS = 1e-5

_T = 64


def _silu(v):
    return v * jax.nn.sigmoid(v)


def _conv2d(v, w, b=None):
    out = jax.lax.conv_general_dilated(
        v, w, (1, 1), "SAME", dimension_numbers=("NCHW", "OIHW", "NCHW"))
    if b is not None:
        out = out + b[None, :, None, None]
    return out


def _scan_all_body(Bsz, T, N, d):
    G = 2 * Bsz * 2

    def body(xf_ref, xb_ref, wm_ref, wb_ref, wc_ref, a_ref, bias_ref, rev_ref,
             yf_ref, yb_ref, h_ref, da_ref, hb_ref):
        c = pl.program_id(0)

        @pl.when(c == 0)
        def _():
            h_ref[...] = jnp.zeros_like(h_ref)

        cc_all = [None] * G
        for dirn in range(2):
            x_ref = xf_ref if dirn == 0 else xb_ref
            for b in range(Bsz):
                X = x_ref[b]
                if dirn == 1:
                    X = jnp.dot(rev_ref[...], X,
                                preferred_element_type=jnp.float32)
                for j in range(2):
                    k = dirn * 2 + j
                    g = (dirn * Bsz + b) * 2 + j
                    sl = slice(g * N, (g + 1) * N)
                    raw = jnp.dot(X, wm_ref[k],
                                  preferred_element_type=jnp.float32)
                    zb = raw + bias_ref[k]
                    delta = (jnp.maximum(zb, 0.0)
                             + jnp.log1p(jnp.exp(-jnp.abs(zb))))
                    Bc = jnp.dot(X, wb_ref[k],
                                 preferred_element_type=jnp.float32)
                    Cc = jnp.dot(X, wc_ref[k],
                                 preferred_element_type=jnp.float32)
                    Gx = delta * X
                    da_ref[:, sl, :] = jnp.exp(
                        delta[:, None, :] * a_ref[k][None, :, :])
                    hb_ref[:, sl, :] = Bc[:, :, None] * Gx[:, None, :]
                    cc_all[g] = Cc

        def step(t, h):
            h = h * da_ref[t] + hb_ref[t]
            hb_ref[t] = h
            return h

        h_ref[...] = jax.lax.fori_loop(0, T, step, h_ref[...])

        for dirn in range(2):
            y_ref = yf_ref if dirn == 0 else yb_ref
            for b in range(Bsz):
                y = jnp.zeros((T, d), jnp.float32)
                for j in range(2):
                    g = (dirn * Bsz + b) * 2 + j
                    sl = slice(g * N, (g + 1) * N)
                    y = y + jnp.sum(hb_ref[:, sl, :]
                                    * cc_all[g][:, :, None], axis=1)
                if dirn == 1:
                    y = jnp.dot(rev_ref[...], y,
                                preferred_element_type=jnp.float32)
                y_ref[b] = y

    return body


def _scan_all(vals, M4, WB4, WC4, A4, bias4):
    Bsz, Lp, d = vals.shape
    T = _T
    NC = Lp // T
    N = _D_STATE
    G = 2 * Bsz * 2

    def fmap(c):
        return (0, c, 0)

    def bmap(c):
        return (0, NC - 1 - c, 0)

    def wmap(c):
        return (0, 0, 0)

    yf, yb = pl.pallas_call(
        _scan_all_body(Bsz, T, N, d),
        grid=(NC,),
        in_specs=[
            pl.BlockSpec((Bsz, T, d), fmap),
            pl.BlockSpec((Bsz, T, d), bmap),
            pl.BlockSpec((4, d, d), wmap),
            pl.BlockSpec((4, d, N), wmap),
            pl.BlockSpec((4, d, N), wmap),
            pl.BlockSpec((4, N, d), wmap),
            pl.BlockSpec((4, 1, d), wmap),
            pl.BlockSpec((T, T), lambda c: (0, 0)),
        ],
        out_specs=[
            pl.BlockSpec((Bsz, T, d), fmap),
            pl.BlockSpec((Bsz, T, d), bmap),
        ],
        out_shape=[
            jax.ShapeDtypeStruct((Bsz, Lp, d), jnp.float32),
            jax.ShapeDtypeStruct((Bsz, Lp, d), jnp.float32),
        ],
        scratch_shapes=[
            pltpu.VMEM((G * N, d), jnp.float32),
            pltpu.VMEM((T, G * N, d), jnp.float32),
            pltpu.VMEM((T, G * N, d), jnp.float32),
        ],
        compiler_params=pltpu.CompilerParams(
            dimension_semantics=("arbitrary",)),
    )(vals, vals, M4, WB4, WC4, A4, bias4,
      jnp.eye(T, dtype=jnp.float32)[::-1])
    return yf, yb


def _select_mask(xp):
    B, C, H, W = xp.shape
    gh, gw = H // _SIZE, W // _SIZE
    density = jax.nn.sigmoid(xp.mean(axis=1))
    density = density.reshape(B, gh, _SIZE, gw, _SIZE).mean(axis=(2, 4))
    sf = density.reshape(B, gh * gw)
    ths = []
    th = 0.3
    while th >= 0:
        ths.append(th)
        th = round(th - 0.05, 2)
    thr_arr = jnp.asarray(ths, dtype=jnp.float32)
    anys = sf[:, None, :] > thr_arr[None, :, None]
    anys = anys.any(axis=2)
    found = anys.any(axis=1)
    th_sel = thr_arr[jnp.argmax(anys, axis=1)]
    m = (sf > th_sel[:, None]).astype(jnp.float32)
    top10 = jax.lax.top_k(sf, 10)[0][:, -1]
    m_cap = (sf >= top10[:, None]).astype(jnp.float32)
    m = jnp.where(m.sum(axis=1, keepdims=True) > 10, m_cap, m)
    fallback = (sf >= sf.max(axis=1, keepdims=True)).astype(jnp.float32)
    return jnp.where(found[:, None], m, fallback)


def _build_indices(mask_cells, H, W, max_n):
    B = mask_cells.shape[0]
    gh, gw = H // _SIZE, W // _SIZE
    sel = mask_cells.reshape(B, gh, gw)
    sel = jnp.repeat(jnp.repeat(sel, _SIZE, axis=1), _SIZE, axis=2)
    sel = sel.reshape(B, H * W) > 0
    HW = H * W
    ar = jnp.arange(HW, dtype=jnp.int32)
    csum = jnp.cumsum(sel.astype(jnp.int32), axis=1)
    total = csum[:, -1]
    keep = sel & (csum <= max_n)
    keep = jnp.where((total > 0)[:, None], keep, (ar < 16)[None, :])
    kcs = jnp.cumsum(keep.astype(jnp.int32), axis=1)
    n_valid = kcs[:, -1]
    rank = jnp.where(keep, kcs - 1, n_valid[:, None] + ar[None, :] - kcs)

    def scat_one(rank_b):
        return jnp.zeros((max_n,), jnp.int32).at[rank_b].set(ar, mode="drop")

    idx = jax.vmap(scat_one)(rank)
    return idx, rank, keep, n_valid


def kernel(x, in_w, in_b, x_proj_weight, dt_projs_weight, dt_projs_bias,
           A_logs, Ds, proj_d_w, ln_g, ln_b, out_w):
    xact = _silu(_conv2d(x, in_w, in_b))
    xp, z = jnp.split(xact, 2, axis=1)
    B, D, H, W = xp.shape
    HW = H * W
    max_n = int(HW * _TOPK)
    Lp = ((max_n + _T - 1) // _T) * _T

    mask_cells = _select_mask(xp)
    idx, rank, keep, n_valid = _build_indices(mask_cells, H, W, max_n)
    validp = jnp.arange(Lp, dtype=jnp.int32)[None, :] < n_valid[:, None]

    xs_flat = xp.reshape(B, D, HW)
    idx_pad = jnp.pad(idx, ((0, 0), (0, Lp - max_n)))
    vals = jnp.take_along_axis(xs_flat, idx_pad[:, None, :], axis=2)
    vals = vals * validp[:, None, :].astype(vals.dtype)
    vals_t = jnp.swapaxes(vals, 1, 2)

    Ds_full = _silu(Ds @ proj_d_w.T)
    Ds_t = Ds_full[:_KG * _D_INNER].reshape(_KG, _D_INNER)
    Ds_b = Ds_full[_KG * _D_INNER:]
    dsum = Ds_t.sum(axis=0)

    W6 = x_proj_weight[:, :_DT_RANK, :]
    WB = x_proj_weight[:, _DT_RANK:_DT_RANK + _D_STATE, :]
    WC = x_proj_weight[:, _DT_RANK + _D_STATE:, :]
    M = jnp.einsum("krd,ker->kde", W6, dt_projs_weight)
    WBt = jnp.swapaxes(WB, 1, 2)
    WCt = jnp.swapaxes(WC, 1, 2)
    A = -jnp.exp(A_logs).reshape(_KG, _D_INNER, _D_STATE)
    At = jnp.swapaxes(A, 1, 2)
    bias = dt_projs_bias.reshape(_KG, 1, _D_INNER)

    y_f, y_b = _scan_all(vals_t, M, WBt, WCt, At, bias)

    y_sum = (y_f + y_b + vals_t * dsum[None, None, :]) * 0.5
    y_comb = jnp.swapaxes(y_sum[:, :max_n, :], 1, 2)

    base = Ds_b[None, :, None] * xs_flat
    valid = validp[:, :max_n]
    idx_m = jnp.where(valid, idx, HW)

    def scat_one(base_b, idx_b, vals_b):
        return base_b.at[:, idx_b].set(vals_b, mode="drop")

    out_flat = jax.vmap(scat_one)(base, idx_m, y_comb)
    out_y = out_flat.reshape(B, D, H, W)

    mu = out_y.mean(axis=1, keepdims=True)
    var = ((out_y - mu) ** 2).mean(axis=1, keepdims=True)
    oy = (out_y - mu) * jax.lax.rsqrt(var + _LN_EPS)
    oy = oy * ln_g[None, :, None, None] + ln_b[None, :, None, None]
    y = oy * z
    return _conv2d(y, out_w, None)
